# baseline (device time: 52479 ns/iter reference)
import jax
import jax.numpy as jnp
from jax import lax
from jax.experimental import pallas as pl
from jax.experimental.pallas import tpu as pltpu

N_X = 2
N_Y = 2
CHUNKS_PER_B = 4


def kernel(O, Wo):
    B, S, H, D = O.shape
    K = H * D
    N = Wo.shape[1]
    Nc = N // N_Y
    S_half = S // N_X
    R = S_half // CHUNKS_PER_B
    NCHUNK = B * CHUNKS_PER_B

    O3 = O.reshape(B, S, K)

    def body(o_hbm, w_ref, out_ref,
             o_peer, o_my, xsend, xrecv, yrecv,
             opeer_sems, omy_sems,
             xsend_sems, xrecv_sems, ysend_sems, yrecv_sems):
        my_x = lax.axis_index("x")
        my_y = lax.axis_index("y")
        xpeer = (1 - my_x, my_y)
        ypeer = (my_x, 1 - my_y)

        peer_start0 = (1 - my_x) * S_half
        my_start0 = my_x * S_half

        o_dmas = []
        for c in range(NCHUNK):
            b, row = divmod(c, CHUNKS_PER_B)
            dma = pltpu.make_async_copy(
                o_hbm.at[b, pl.ds(peer_start0 + row * R, R), :],
                o_peer.at[c],
                opeer_sems.at[c],
            )
            dma.start()
            o_dmas.append(dma)
        omy_dmas = []
        for b in range(B):
            dma = pltpu.make_async_copy(
                o_hbm.at[b, pl.ds(my_start0, S_half), :],
                o_my.at[b],
                omy_sems.at[b],
            )
            dma.start()
            omy_dmas.append(dma)

        barrier_sem = pltpu.get_barrier_semaphore()
        for nbr in (xpeer, ypeer):
            pl.semaphore_signal(
                barrier_sem, inc=1,
                device_id=nbr, device_id_type=pl.DeviceIdType.MESH,
            )
        pl.semaphore_wait(barrier_sem, 2)

        w_bf = w_ref[...].astype(jnp.bfloat16)

        x_rdmas = []
        for c in range(NCHUNK):
            b, row = divmod(c, CHUNKS_PER_B)
            o_dmas[c].wait()
            o_bf = o_peer[c].astype(jnp.bfloat16)

            @pl.when(my_y == 0)
            def _(o_bf=o_bf, c=c):
                xsend[c, :, :] = jnp.dot(
                    o_bf, w_bf[:, 0:Nc], preferred_element_type=jnp.float32
                ).astype(jnp.bfloat16)

            @pl.when(my_y == 1)
            def _(o_bf=o_bf, c=c):
                xsend[c, :, :] = jnp.dot(
                    o_bf, w_bf[:, Nc:N], preferred_element_type=jnp.float32
                ).astype(jnp.bfloat16)

            rdma = pltpu.make_async_remote_copy(
                src_ref=xsend.at[c],
                dst_ref=xrecv.at[c],
                send_sem=xsend_sems.at[c],
                recv_sem=xrecv_sems.at[c],
                device_id=xpeer,
                device_id_type=pl.DeviceIdType.MESH,
            )
            rdma.start()
            x_rdmas.append(rdma)

        def make_relay(c):
            return pltpu.make_async_remote_copy(
                src_ref=xrecv.at[c],
                dst_ref=yrecv.at[c],
                send_sem=ysend_sems.at[c],
                recv_sem=yrecv_sems.at[c],
                device_id=ypeer,
                device_id_type=pl.DeviceIdType.MESH,
            )

        def add_xrecv(c):
            b, row = divmod(c, CHUNKS_PER_B)

            @pl.when(my_y == 0)
            def _():
                out_ref[b, row * R:(row + 1) * R, 0:Nc] += xrecv[c]

            @pl.when(my_y == 1)
            def _():
                out_ref[b, row * R:(row + 1) * R, Nc:N] += xrecv[c]

        y_rdmas = [None] * NCHUNK
        x_rdmas[0].wait_recv()
        y_rdmas[0] = make_relay(0)
        y_rdmas[0].start()

        omy_dmas[0].wait()
        o_bf = o_my[0].astype(jnp.bfloat16)
        out_ref[0, :, :] = jnp.dot(
            o_bf, w_bf, preferred_element_type=jnp.float32
        ).astype(jnp.bfloat16)

        add_xrecv(0)
        for c in range(1, CHUNKS_PER_B):
            x_rdmas[c].wait_recv()
            y_rdmas[c] = make_relay(c)
            y_rdmas[c].start()
            add_xrecv(c)

        omy_dmas[1].wait()
        o_bf = o_my[1].astype(jnp.bfloat16)
        out_ref[1, :, :] = jnp.dot(
            o_bf, w_bf, preferred_element_type=jnp.float32
        ).astype(jnp.bfloat16)

        for c in range(CHUNKS_PER_B, NCHUNK):
            x_rdmas[c].wait_recv()
            y_rdmas[c] = make_relay(c)
            y_rdmas[c].start()
            add_xrecv(c)

        for c in range(NCHUNK):
            b, row = divmod(c, CHUNKS_PER_B)
            y_rdmas[c].wait_recv()

            @pl.when(my_y == 0)
            def _(c=c, b=b, row=row):
                out_ref[b, row * R:(row + 1) * R, Nc:N] += yrecv[c]

            @pl.when(my_y == 1)
            def _(c=c, b=b, row=row):
                out_ref[b, row * R:(row + 1) * R, 0:Nc] += yrecv[c]

        for c in range(NCHUNK):
            x_rdmas[c].wait_send()
            y_rdmas[c].wait_send()

    return pl.pallas_call(
        body,
        out_shape=jax.ShapeDtypeStruct((B, S_half, N), jnp.bfloat16),
        in_specs=[
            pl.BlockSpec(memory_space=pl.ANY),
            pl.BlockSpec(memory_space=pltpu.VMEM),
        ],
        out_specs=pl.BlockSpec(memory_space=pltpu.VMEM),
        scratch_shapes=[
            pltpu.VMEM((NCHUNK, R, K), jnp.float32),
            pltpu.VMEM((B, S_half, K), jnp.float32),
            pltpu.VMEM((NCHUNK, R, Nc), jnp.bfloat16),
            pltpu.VMEM((NCHUNK, R, Nc), jnp.bfloat16),
            pltpu.VMEM((NCHUNK, R, Nc), jnp.bfloat16),
            pltpu.SemaphoreType.DMA((NCHUNK,)),
            pltpu.SemaphoreType.DMA((B,)),
            pltpu.SemaphoreType.DMA((NCHUNK,)),
            pltpu.SemaphoreType.DMA((NCHUNK,)),
            pltpu.SemaphoreType.DMA((NCHUNK,)),
            pltpu.SemaphoreType.DMA((NCHUNK,)),
        ],
        compiler_params=pltpu.CompilerParams(
            collective_id=0, vmem_limit_bytes=96 * 1024 * 1024
        ),
    )(O3, Wo)


# device time: 41970 ns/iter; 1.2504x vs baseline; 1.2504x over previous
import jax
import jax.numpy as jnp
from jax import lax
from jax.experimental import pallas as pl
from jax.experimental.pallas import tpu as pltpu

N_X = 2
N_Y = 2
CHUNKS_PER_B = 4


def kernel(O, Wo):
    B, S, H, D = O.shape
    K = H * D
    N = Wo.shape[1]
    Nc = N // N_Y
    S_half = S // N_X
    R = S_half // CHUNKS_PER_B
    NCHUNK = B * CHUNKS_PER_B

    O3 = O.reshape(B, S, K)

    def body(o_ref, w_ref, out_ref,
             xsend, xrecv, yrecv,
             xsend_sems, xrecv_sems, ysend_sems, yrecv_sems):
        my_x = lax.axis_index("x")
        my_y = lax.axis_index("y")
        xpeer = (1 - my_x, my_y)
        ypeer = (my_x, 1 - my_y)

        barrier_sem = pltpu.get_barrier_semaphore()
        for nbr in (xpeer, ypeer):
            pl.semaphore_signal(
                barrier_sem, inc=1,
                device_id=nbr, device_id_type=pl.DeviceIdType.MESH,
            )

        w_bf = w_ref[...].astype(jnp.bfloat16)
        peer_start = (1 - my_x) * S_half
        my_start = my_x * S_half

        def compute_chunk(c):
            b, row = divmod(c, CHUNKS_PER_B)
            o_bf = o_ref[b, pl.ds(peer_start + row * R, R), :].astype(
                jnp.bfloat16
            )

            @pl.when(my_y == 0)
            def _():
                xsend[c, :, :] = jnp.dot(
                    o_bf, w_bf[:, 0:Nc], preferred_element_type=jnp.float32
                ).astype(jnp.bfloat16)

            @pl.when(my_y == 1)
            def _():
                xsend[c, :, :] = jnp.dot(
                    o_bf, w_bf[:, Nc:N], preferred_element_type=jnp.float32
                ).astype(jnp.bfloat16)

        compute_chunk(0)
        pl.semaphore_wait(barrier_sem, 2)

        x_rdmas = []
        for c in range(NCHUNK):
            if c > 0:
                compute_chunk(c)
            rdma = pltpu.make_async_remote_copy(
                src_ref=xsend.at[c],
                dst_ref=xrecv.at[c],
                send_sem=xsend_sems.at[c],
                recv_sem=xrecv_sems.at[c],
                device_id=xpeer,
                device_id_type=pl.DeviceIdType.MESH,
            )
            rdma.start()
            x_rdmas.append(rdma)

        def make_relay(c):
            return pltpu.make_async_remote_copy(
                src_ref=xrecv.at[c],
                dst_ref=yrecv.at[c],
                send_sem=ysend_sems.at[c],
                recv_sem=yrecv_sems.at[c],
                device_id=ypeer,
                device_id_type=pl.DeviceIdType.MESH,
            )

        def add_xrecv(c):
            b, row = divmod(c, CHUNKS_PER_B)

            @pl.when(my_y == 0)
            def _():
                out_ref[b, row * R:(row + 1) * R, 0:Nc] += xrecv[c]

            @pl.when(my_y == 1)
            def _():
                out_ref[b, row * R:(row + 1) * R, Nc:N] += xrecv[c]

        y_rdmas = [None] * NCHUNK
        x_rdmas[0].wait_recv()
        y_rdmas[0] = make_relay(0)
        y_rdmas[0].start()

        o_bf = o_ref[0, pl.ds(my_start, S_half), :].astype(jnp.bfloat16)
        out_ref[0, :, :] = jnp.dot(
            o_bf, w_bf, preferred_element_type=jnp.float32
        ).astype(jnp.bfloat16)

        add_xrecv(0)
        for c in range(1, CHUNKS_PER_B):
            x_rdmas[c].wait_recv()
            y_rdmas[c] = make_relay(c)
            y_rdmas[c].start()
            add_xrecv(c)

        o_bf = o_ref[1, pl.ds(my_start, S_half), :].astype(jnp.bfloat16)
        out_ref[1, :, :] = jnp.dot(
            o_bf, w_bf, preferred_element_type=jnp.float32
        ).astype(jnp.bfloat16)

        for c in range(CHUNKS_PER_B, NCHUNK):
            x_rdmas[c].wait_recv()
            y_rdmas[c] = make_relay(c)
            y_rdmas[c].start()
            add_xrecv(c)

        for c in range(NCHUNK):
            b, row = divmod(c, CHUNKS_PER_B)
            y_rdmas[c].wait_recv()

            @pl.when(my_y == 0)
            def _(c=c, b=b, row=row):
                out_ref[b, row * R:(row + 1) * R, Nc:N] += yrecv[c]

            @pl.when(my_y == 1)
            def _(c=c, b=b, row=row):
                out_ref[b, row * R:(row + 1) * R, 0:Nc] += yrecv[c]

        for c in range(NCHUNK):
            x_rdmas[c].wait_send()
            y_rdmas[c].wait_send()

    return pl.pallas_call(
        body,
        out_shape=jax.ShapeDtypeStruct((B, S_half, N), jnp.bfloat16),
        in_specs=[
            pl.BlockSpec(memory_space=pltpu.VMEM),
            pl.BlockSpec(memory_space=pltpu.VMEM),
        ],
        out_specs=pl.BlockSpec(memory_space=pltpu.VMEM),
        scratch_shapes=[
            pltpu.VMEM((NCHUNK, R, Nc), jnp.bfloat16),
            pltpu.VMEM((NCHUNK, R, Nc), jnp.bfloat16),
            pltpu.VMEM((NCHUNK, R, Nc), jnp.bfloat16),
            pltpu.SemaphoreType.DMA((NCHUNK,)),
            pltpu.SemaphoreType.DMA((NCHUNK,)),
            pltpu.SemaphoreType.DMA((NCHUNK,)),
            pltpu.SemaphoreType.DMA((NCHUNK,)),
        ],
        compiler_params=pltpu.CompilerParams(collective_id=0),
    )(O3, Wo)
